# Initial kernel scaffold; baseline (speedup 1.0000x reference)
import jax
import jax.numpy as jnp
from jax.experimental import pallas as pl


def kernel(xyz, points, W1, b1, gamma1, beta1, W2, b2, gamma2, beta2):
    raise NotImplementedError("write your pallas kernel here")


# speedup vs baseline: 12.7029x; 12.7029x over previous
import functools

import jax
import jax.numpy as jnp
from jax import lax
from jax.experimental import pallas as pl
from jax.experimental.pallas import tpu as pltpu
from jax.experimental.pallas import tpu_sc as plsc

_B, _N, _S, _K, _D = 8, 4096, 1024, 16, 128
_R = _B * _S * _K
_BS = _B * _S
_TBL = _B * _N
_TC = 144
_C1 = 256
_C2 = 256
_EPS = 1e-5
_SBLK = 256
_RBLK = 2048
_GBLK = _RBLK // _K
_BIG = 1e30


def _fps_kernel(x_ref, y_ref, z_ref, idx_ref, nx_ref, ny_ref, nz_ref,
                bi_ref, bx_ref, by_ref, bz_ref):
    x = x_ref[...]
    y = y_ref[...]
    z = z_ref[...]
    lane = lax.broadcasted_iota(jnp.int32, (_B, _N), 1)
    lane128 = lax.broadcasted_iota(jnp.int32, (_B, 128), 1)

    def body(i, carry):
        dist, f = carry
        csel = lane == f
        cx = jnp.sum(jnp.where(csel, x, 0.0), axis=1, keepdims=True)
        cy = jnp.sum(jnp.where(csel, y, 0.0), axis=1, keepdims=True)
        cz = jnp.sum(jnp.where(csel, z, 0.0), axis=1, keepdims=True)
        sel = lane128 == i
        seli = sel.astype(jnp.int32)
        self_ = sel.astype(jnp.float32)
        bi_ref[...] = bi_ref[...] + f * seli
        bx_ref[...] = bx_ref[...] + cx * self_
        by_ref[...] = by_ref[...] + cy * self_
        bz_ref[...] = bz_ref[...] + cz * self_
        dx = x - cx
        dy = y - cy
        dz = z - cz
        d = dx * dx + dy * dy + dz * dz
        dist = jnp.minimum(dist, d)
        m = jnp.max(dist, axis=1, keepdims=True)
        f2 = jnp.min(jnp.where(dist == m, lane, _N), axis=1, keepdims=True)
        return dist, f2

    dist = jnp.full((_B, _N), 1e10, dtype=jnp.float32)
    f = jnp.zeros((_B, 1), dtype=jnp.int32)
    for o in range(_S // 128):
        bi_ref[...] = jnp.zeros((_B, 128), dtype=jnp.int32)
        bx_ref[...] = jnp.zeros((_B, 128), dtype=jnp.float32)
        by_ref[...] = jnp.zeros((_B, 128), dtype=jnp.float32)
        bz_ref[...] = jnp.zeros((_B, 128), dtype=jnp.float32)
        dist, f = lax.fori_loop(0, 128, body, (dist, f))
        idx_ref[:, o * 128:(o + 1) * 128] = bi_ref[...]
        nx_ref[:, o * 128:(o + 1) * 128] = bx_ref[...]
        ny_ref[:, o * 128:(o + 1) * 128] = by_ref[...]
        nz_ref[:, o * 128:(o + 1) * 128] = bz_ref[...]


def _run_fps(x, y, z):
    return pl.pallas_call(
        _fps_kernel,
        out_shape=(
            jax.ShapeDtypeStruct((_B, _S), jnp.int32),
            jax.ShapeDtypeStruct((_B, _S), jnp.float32),
            jax.ShapeDtypeStruct((_B, _S), jnp.float32),
            jax.ShapeDtypeStruct((_B, _S), jnp.float32),
        ),
        scratch_shapes=[
            pltpu.VMEM((_B, 128), jnp.int32),
            pltpu.VMEM((_B, 128), jnp.float32),
            pltpu.VMEM((_B, 128), jnp.float32),
            pltpu.VMEM((_B, 128), jnp.float32),
        ],
    )(x, y, z)


def _knn_kernel(nxyz_ref, xt_ref, idx_ref, dist_ref):
    b = pl.program_id(0)
    src = nxyz_ref[0]
    dst = xt_ref[0]
    rn = jnp.sum(src * src, axis=1, keepdims=True)
    cn = jnp.sum(dst * dst, axis=0, keepdims=True)
    m = lax.dot_general(src, dst, (((1,), (0,)), ((), ())),
                        preferred_element_type=jnp.float32)
    dist_ref[...] = rn + cn - 2.0 * m
    coli = lax.broadcasted_iota(jnp.int32, (_SBLK, _N), 1)
    base = b * _N
    for k in range(_K):
        d = dist_ref[...]
        mn = jnp.min(d, axis=1, keepdims=True)
        j = jnp.min(jnp.where(d == mn, coli, _N), axis=1, keepdims=True)
        idx_ref[0, :, k:k + 1] = j + base
        dist_ref[...] = jnp.where(coli == j, _BIG, d)


def _run_knn(new_xyz, xt):
    return pl.pallas_call(
        _knn_kernel,
        grid=(_B, _S // _SBLK),
        in_specs=[
            pl.BlockSpec((1, _SBLK, 3), lambda b, j: (b, j, 0)),
            pl.BlockSpec((1, 3, _N), lambda b, j: (b, 0, 0)),
        ],
        out_specs=pl.BlockSpec((1, _SBLK, _K), lambda b, j: (b, j, 0)),
        out_shape=jax.ShapeDtypeStruct((_B, _S, _K), jnp.int32),
        scratch_shapes=[pltpu.VMEM((_SBLK, _N), jnp.float32)],
    )(new_xyz, xt)


def _sc_gather(table, gidx):
    info = plsc.get_sparse_core_info()
    nw = info.num_cores * info.num_subcores
    rows_per_w = _R // nw
    chunk = 128
    nchunk = rows_per_w // chunk
    mesh = plsc.VectorSubcoreMesh(core_axis_name="c", subcore_axis_name="s")

    @functools.partial(
        pl.kernel,
        mesh=mesh,
        out_type=jax.ShapeDtypeStruct((_R, _C1), jnp.float32),
        scratch_types=[
            pltpu.VMEM((chunk,), jnp.int32),
            pltpu.VMEM((chunk, _C1), jnp.float32),
            pltpu.SemaphoreType.DMA,
        ],
    )
    def k(table_hbm, gidx_hbm, out_hbm, idx_v, rows_v, sem):
        wid = lax.axis_index("s") * info.num_cores + lax.axis_index("c")
        base = wid * rows_per_w

        def body(c, carry):
            off = base + c * chunk
            pltpu.sync_copy(gidx_hbm.at[pl.ds(off, chunk)], idx_v)
            pltpu.async_copy(table_hbm.at[idx_v], rows_v, sem).wait()
            pltpu.sync_copy(rows_v, out_hbm.at[pl.ds(off, chunk)])
            return carry

        lax.fori_loop(0, nchunk, body, 0)

    return k(table, gidx)


def _ptrans_kernel(xyz_ref, pts_ref, w1xt_ref, w1ft_ref, b1_ref, o_ref):
    h = lax.dot_general(pts_ref[...], w1ft_ref[...], (((1,), (0,)), ((), ())),
                        preferred_element_type=jnp.float32)
    h = h + lax.dot_general(xyz_ref[...], w1xt_ref[...],
                            (((1,), (0,)), ((), ())),
                            preferred_element_type=jnp.float32)
    o_ref[...] = h + b1_ref[...]


def _run_ptrans(xyz_flat, pts_flat, w1xt, w1ft, b1):
    blk = 4096
    return pl.pallas_call(
        _ptrans_kernel,
        grid=(_TBL // blk,),
        in_specs=[
            pl.BlockSpec((blk, 3), lambda i: (i, 0)),
            pl.BlockSpec((blk, _D), lambda i: (i, 0)),
            pl.BlockSpec((3, _C1), lambda i: (0, 0)),
            pl.BlockSpec((_D, _C1), lambda i: (0, 0)),
            pl.BlockSpec((1, _C1), lambda i: (0, 0)),
        ],
        out_specs=pl.BlockSpec((blk, _C1), lambda i: (i, 0)),
        out_shape=jax.ShapeDtypeStruct((_TBL, _C1), jnp.float32),
    )(xyz_flat, pts_flat, w1xt, w1ft, b1)


def _stats1_kernel(g_ref, w1xt_ref, nx_ref, s_ref, ss_ref):
    i = pl.program_id(0)
    nxw = lax.dot_general(nx_ref[...], w1xt_ref[...], (((1,), (0,)), ((), ())),
                          preferred_element_type=jnp.float32)
    h3 = g_ref[...].reshape(_GBLK, _K, _C1) - nxw[:, None, :]
    ps = jnp.sum(h3, axis=(0, 1)).reshape(1, _C1)
    pss = jnp.sum(h3 * h3, axis=(0, 1)).reshape(1, _C1)

    @pl.when(i == 0)
    def _():
        s_ref[...] = ps
        ss_ref[...] = pss

    @pl.when(i > 0)
    def _():
        s_ref[...] += ps
        ss_ref[...] += pss


def _run_stats1(g, w1xt, nxf):
    return pl.pallas_call(
        _stats1_kernel,
        grid=(_R // _RBLK,),
        in_specs=[
            pl.BlockSpec((_RBLK, _C1), lambda i: (i, 0)),
            pl.BlockSpec((3, _C1), lambda i: (0, 0)),
            pl.BlockSpec((_GBLK, 3), lambda i: (i, 0)),
        ],
        out_specs=(
            pl.BlockSpec((1, _C1), lambda i: (0, 0)),
            pl.BlockSpec((1, _C1), lambda i: (0, 0)),
        ),
        out_shape=(
            jax.ShapeDtypeStruct((1, _C1), jnp.float32),
            jax.ShapeDtypeStruct((1, _C1), jnp.float32),
        ),
    )(g, w1xt, nxf)


def _mlp2_kernel(g_ref, w1xt_ref, nx_ref, s1_ref, ss1_ref, g1_ref, be1_ref,
                 w2t_ref, b2_ref, hmax_ref, hmin_ref, s2_ref, ss2_ref):
    i = pl.program_id(0)
    rinv = jnp.float32(1.0 / _R)
    m1 = s1_ref[...] * rinv
    v1 = ss1_ref[...] * rinv - m1 * m1
    sc1 = g1_ref[...] / jnp.sqrt(v1 + _EPS)
    nxw = lax.dot_general(nx_ref[...], w1xt_ref[...], (((1,), (0,)), ((), ())),
                          preferred_element_type=jnp.float32)
    h1 = (g_ref[...].reshape(_GBLK, _K, _C1)
          - nxw[:, None, :]).reshape(_RBLK, _C1)
    x = jnp.maximum((h1 - m1) * sc1 + be1_ref[...], 0.0)
    h2 = lax.dot_general(x, w2t_ref[...], (((1,), (0,)), ((), ())),
                         preferred_element_type=jnp.float32) + b2_ref[...]
    ps = jnp.sum(h2, axis=0, keepdims=True)
    pss = jnp.sum(h2 * h2, axis=0, keepdims=True)
    h3 = h2.reshape(_GBLK, _K, _C2)
    hmax_ref[...] = jnp.max(h3, axis=1)
    hmin_ref[...] = jnp.min(h3, axis=1)

    @pl.when(i == 0)
    def _():
        s2_ref[...] = ps
        ss2_ref[...] = pss

    @pl.when(i > 0)
    def _():
        s2_ref[...] += ps
        ss2_ref[...] += pss


def _run_mlp2(g, w1xt, nxf, s1, ss1, g1, be1, w2t, b2):
    return pl.pallas_call(
        _mlp2_kernel,
        grid=(_R // _RBLK,),
        in_specs=[
            pl.BlockSpec((_RBLK, _C1), lambda i: (i, 0)),
            pl.BlockSpec((3, _C1), lambda i: (0, 0)),
            pl.BlockSpec((_GBLK, 3), lambda i: (i, 0)),
            pl.BlockSpec((1, _C1), lambda i: (0, 0)),
            pl.BlockSpec((1, _C1), lambda i: (0, 0)),
            pl.BlockSpec((1, _C1), lambda i: (0, 0)),
            pl.BlockSpec((1, _C1), lambda i: (0, 0)),
            pl.BlockSpec((_C1, _C2), lambda i: (0, 0)),
            pl.BlockSpec((1, _C2), lambda i: (0, 0)),
        ],
        out_specs=(
            pl.BlockSpec((_GBLK, _C2), lambda i: (i, 0)),
            pl.BlockSpec((_GBLK, _C2), lambda i: (i, 0)),
            pl.BlockSpec((1, _C2), lambda i: (0, 0)),
            pl.BlockSpec((1, _C2), lambda i: (0, 0)),
        ),
        out_shape=(
            jax.ShapeDtypeStruct((_BS, _C2), jnp.float32),
            jax.ShapeDtypeStruct((_BS, _C2), jnp.float32),
            jax.ShapeDtypeStruct((1, _C2), jnp.float32),
            jax.ShapeDtypeStruct((1, _C2), jnp.float32),
        ),
    )(g, w1xt, nxf, s1, ss1, g1, be1, w2t, b2)


def _final_kernel(hmax_ref, hmin_ref, s2_ref, ss2_ref, g2_ref, be2_ref, o_ref):
    rinv = jnp.float32(1.0 / _R)
    m2 = s2_ref[...] * rinv
    v2 = ss2_ref[...] * rinv - m2 * m2
    g = g2_ref[...]
    sc2 = g / jnp.sqrt(v2 + _EPS)
    h = jnp.where(g >= 0.0, hmax_ref[...], hmin_ref[...])
    o_ref[...] = jnp.maximum((h - m2) * sc2 + be2_ref[...], 0.0)


def _run_final(hmax, hmin, s2, ss2, g2, be2):
    blk = 2048
    return pl.pallas_call(
        _final_kernel,
        grid=(_BS // blk,),
        in_specs=[
            pl.BlockSpec((blk, _C2), lambda i: (i, 0)),
            pl.BlockSpec((blk, _C2), lambda i: (i, 0)),
            pl.BlockSpec((1, _C2), lambda i: (0, 0)),
            pl.BlockSpec((1, _C2), lambda i: (0, 0)),
            pl.BlockSpec((1, _C2), lambda i: (0, 0)),
            pl.BlockSpec((1, _C2), lambda i: (0, 0)),
        ],
        out_specs=pl.BlockSpec((blk, _C2), lambda i: (i, 0)),
        out_shape=jax.ShapeDtypeStruct((_BS, _C2), jnp.float32),
    )(hmax, hmin, s2, ss2, g2, be2)


_gather_rows = _sc_gather


def kernel(xyz, points, W1, b1, gamma1, beta1, W2, b2, gamma2, beta2):
    xt = jnp.swapaxes(xyz, 1, 2)
    x, y, z = xt[:, 0], xt[:, 1], xt[:, 2]
    _, nx, ny, nz = _run_fps(x, y, z)
    new_xyz = jnp.stack([nx, ny, nz], axis=-1)

    gidx = _run_knn(new_xyz, xt)

    w1ft = W1[:, 3:].T
    w1xt = W1[:, :3].T
    xw = _run_ptrans(xyz.reshape(_TBL, 3), points.reshape(_TBL, _D),
                     w1xt, w1ft, b1.reshape(1, _C1))
    g = _gather_rows(xw, gidx.reshape(_R))

    nxf = new_xyz.reshape(_BS, 3)
    s1, ss1 = _run_stats1(g, w1xt, nxf)
    hmax, hmin, s2, ss2 = _run_mlp2(g, w1xt, nxf, s1, ss1,
                                    gamma1.reshape(1, _C1),
                                    beta1.reshape(1, _C1), W2.T,
                                    b2.reshape(1, _C2))

    out = _run_final(hmax, hmin, s2, ss2, gamma2.reshape(1, _C2),
                     beta2.reshape(1, _C2))
    return new_xyz, out.reshape(_B, _S, _C2)


# speedup vs baseline: 13.5399x; 1.0659x over previous
import functools

import jax
import jax.numpy as jnp
from jax import lax
from jax.experimental import pallas as pl
from jax.experimental.pallas import tpu as pltpu
from jax.experimental.pallas import tpu_sc as plsc

_B, _N, _S, _K, _D = 8, 4096, 1024, 16, 128
_R = _B * _S * _K
_BS = _B * _S
_TBL = _B * _N
_TC = 144
_C1 = 256
_C2 = 256
_EPS = 1e-5
_SBLK = 256
_RBLK = 2048
_GBLK = _RBLK // _K
_BIG = 1e30


def _fps_kernel(x_ref, y_ref, z_ref, idx_ref, nx_ref, ny_ref, nz_ref,
                bi_ref, bx_ref, by_ref, bz_ref):
    x = x_ref[...]
    y = y_ref[...]
    z = z_ref[...]
    lane128 = lax.broadcasted_iota(jnp.int32, (_B, 128), 1)
    lanef = lax.broadcasted_iota(
        jnp.int32, (_B, _N), 1).astype(jnp.float32)

    def body(i, carry):
        dist, f, cx, cy, cz = carry
        sel = lane128 == i
        seli = sel.astype(jnp.int32)
        self_ = sel.astype(jnp.float32)
        bi_ref[...] = bi_ref[...] + f * seli
        bx_ref[...] = bx_ref[...] + cx * self_
        by_ref[...] = by_ref[...] + cy * self_
        bz_ref[...] = bz_ref[...] + cz * self_
        dx = x - cx
        dy = y - cy
        dz = z - cz
        d = dx * dx + dy * dy + dz * dz
        dist = jnp.minimum(dist, d)

        td, tl, tx, ty, tz = dist, lanef, x, y, z
        w = _N
        while w > 128:
            h = w // 2
            take = td[:, h:] > td[:, :h]
            td = jnp.where(take, td[:, h:], td[:, :h])
            tl = jnp.where(take, tl[:, h:], tl[:, :h])
            tx = jnp.where(take, tx[:, h:], tx[:, :h])
            ty = jnp.where(take, ty[:, h:], ty[:, :h])
            tz = jnp.where(take, tz[:, h:], tz[:, :h])
            w = h
        m = jnp.max(td, axis=1, keepdims=True)
        fi = jnp.min(jnp.where(td == m, tl, float(_N)),
                     axis=1, keepdims=True)
        s2 = tl == fi
        cx2 = jnp.sum(jnp.where(s2, tx, 0.0), axis=1, keepdims=True)
        cy2 = jnp.sum(jnp.where(s2, ty, 0.0), axis=1, keepdims=True)
        cz2 = jnp.sum(jnp.where(s2, tz, 0.0), axis=1, keepdims=True)
        f2 = fi.astype(jnp.int32)
        return dist, f2, cx2, cy2, cz2

    dist = jnp.full((_B, _N), 1e10, dtype=jnp.float32)
    f = jnp.zeros((_B, 1), dtype=jnp.int32)
    cx = x[:, 0:1]
    cy = y[:, 0:1]
    cz = z[:, 0:1]
    for o in range(_S // 128):
        bi_ref[...] = jnp.zeros((_B, 128), dtype=jnp.int32)
        bx_ref[...] = jnp.zeros((_B, 128), dtype=jnp.float32)
        by_ref[...] = jnp.zeros((_B, 128), dtype=jnp.float32)
        bz_ref[...] = jnp.zeros((_B, 128), dtype=jnp.float32)
        dist, f, cx, cy, cz = lax.fori_loop(0, 128, body,
                                            (dist, f, cx, cy, cz))
        idx_ref[:, o * 128:(o + 1) * 128] = bi_ref[...]
        nx_ref[:, o * 128:(o + 1) * 128] = bx_ref[...]
        ny_ref[:, o * 128:(o + 1) * 128] = by_ref[...]
        nz_ref[:, o * 128:(o + 1) * 128] = bz_ref[...]


def _run_fps(x, y, z):
    return pl.pallas_call(
        _fps_kernel,
        out_shape=(
            jax.ShapeDtypeStruct((_B, _S), jnp.int32),
            jax.ShapeDtypeStruct((_B, _S), jnp.float32),
            jax.ShapeDtypeStruct((_B, _S), jnp.float32),
            jax.ShapeDtypeStruct((_B, _S), jnp.float32),
        ),
        scratch_shapes=[
            pltpu.VMEM((_B, 128), jnp.int32),
            pltpu.VMEM((_B, 128), jnp.float32),
            pltpu.VMEM((_B, 128), jnp.float32),
            pltpu.VMEM((_B, 128), jnp.float32),
        ],
    )(x, y, z)


def _knn_kernel(nxyz_ref, xt_ref, idx_ref, dist_ref):
    b = pl.program_id(0)
    src = nxyz_ref[0]
    dst = xt_ref[0]
    rn = jnp.sum(src * src, axis=1, keepdims=True)
    cn = jnp.sum(dst * dst, axis=0, keepdims=True)
    m = lax.dot_general(src, dst, (((1,), (0,)), ((), ())),
                        preferred_element_type=jnp.float32)
    dist_ref[...] = rn + cn - 2.0 * m
    colf = lax.broadcasted_iota(
        jnp.int32, (_SBLK, _N), 1).astype(jnp.float32)
    base = b * _N
    for k in range(_K):
        d = dist_ref[...]
        td, tl = d, colf
        w = _N
        while w > 128:
            h = w // 2
            take = td[:, h:] < td[:, :h]
            td = jnp.where(take, td[:, h:], td[:, :h])
            tl = jnp.where(take, tl[:, h:], tl[:, :h])
            w = h
        mn = jnp.min(td, axis=1, keepdims=True)
        fj = jnp.min(jnp.where(td == mn, tl, float(_N)),
                     axis=1, keepdims=True)
        j = fj.astype(jnp.int32)
        idx_ref[0, :, k:k + 1] = j + base
        dist_ref[...] = jnp.where(colf == fj, _BIG, d)


def _run_knn(new_xyz, xt):
    return pl.pallas_call(
        _knn_kernel,
        grid=(_B, _S // _SBLK),
        in_specs=[
            pl.BlockSpec((1, _SBLK, 3), lambda b, j: (b, j, 0)),
            pl.BlockSpec((1, 3, _N), lambda b, j: (b, 0, 0)),
        ],
        out_specs=pl.BlockSpec((1, _SBLK, _K), lambda b, j: (b, j, 0)),
        out_shape=jax.ShapeDtypeStruct((_B, _S, _K), jnp.int32),
        scratch_shapes=[pltpu.VMEM((_SBLK, _N), jnp.float32)],
    )(new_xyz, xt)


def _sc_gather(table, gidx):
    info = plsc.get_sparse_core_info()
    nw = info.num_cores * info.num_subcores
    rows_per_w = _R // nw
    chunk = 128
    nchunk = rows_per_w // chunk
    mesh = plsc.VectorSubcoreMesh(core_axis_name="c", subcore_axis_name="s")

    @functools.partial(
        pl.kernel,
        mesh=mesh,
        out_type=jax.ShapeDtypeStruct((_R, _C1), jnp.float32),
        scratch_types=[
            pltpu.VMEM((chunk,), jnp.int32),
            pltpu.VMEM((chunk, _C1), jnp.float32),
            pltpu.SemaphoreType.DMA,
        ],
    )
    def k(table_hbm, gidx_hbm, out_hbm, idx_v, rows_v, sem):
        wid = lax.axis_index("s") * info.num_cores + lax.axis_index("c")
        base = wid * rows_per_w

        def body(c, carry):
            off = base + c * chunk
            pltpu.sync_copy(gidx_hbm.at[pl.ds(off, chunk)], idx_v)
            pltpu.async_copy(table_hbm.at[idx_v], rows_v, sem).wait()
            pltpu.sync_copy(rows_v, out_hbm.at[pl.ds(off, chunk)])
            return carry

        lax.fori_loop(0, nchunk, body, 0)

    return k(table, gidx)


def _ptrans_kernel(xyz_ref, pts_ref, w1xt_ref, w1ft_ref, b1_ref, o_ref):
    h = lax.dot_general(pts_ref[...], w1ft_ref[...], (((1,), (0,)), ((), ())),
                        preferred_element_type=jnp.float32)
    h = h + lax.dot_general(xyz_ref[...], w1xt_ref[...],
                            (((1,), (0,)), ((), ())),
                            preferred_element_type=jnp.float32)
    o_ref[...] = h + b1_ref[...]


def _run_ptrans(xyz_flat, pts_flat, w1xt, w1ft, b1):
    blk = 4096
    return pl.pallas_call(
        _ptrans_kernel,
        grid=(_TBL // blk,),
        in_specs=[
            pl.BlockSpec((blk, 3), lambda i: (i, 0)),
            pl.BlockSpec((blk, _D), lambda i: (i, 0)),
            pl.BlockSpec((3, _C1), lambda i: (0, 0)),
            pl.BlockSpec((_D, _C1), lambda i: (0, 0)),
            pl.BlockSpec((1, _C1), lambda i: (0, 0)),
        ],
        out_specs=pl.BlockSpec((blk, _C1), lambda i: (i, 0)),
        out_shape=jax.ShapeDtypeStruct((_TBL, _C1), jnp.float32),
    )(xyz_flat, pts_flat, w1xt, w1ft, b1)


def _stats1_kernel(g_ref, w1xt_ref, nx_ref, s_ref, ss_ref):
    i = pl.program_id(0)
    nxw = lax.dot_general(nx_ref[...], w1xt_ref[...], (((1,), (0,)), ((), ())),
                          preferred_element_type=jnp.float32)
    h3 = g_ref[...].reshape(_GBLK, _K, _C1) - nxw[:, None, :]
    ps = jnp.sum(h3, axis=(0, 1)).reshape(1, _C1)
    pss = jnp.sum(h3 * h3, axis=(0, 1)).reshape(1, _C1)

    @pl.when(i == 0)
    def _():
        s_ref[...] = ps
        ss_ref[...] = pss

    @pl.when(i > 0)
    def _():
        s_ref[...] += ps
        ss_ref[...] += pss


def _run_stats1(g, w1xt, nxf):
    return pl.pallas_call(
        _stats1_kernel,
        grid=(_R // _RBLK,),
        in_specs=[
            pl.BlockSpec((_RBLK, _C1), lambda i: (i, 0)),
            pl.BlockSpec((3, _C1), lambda i: (0, 0)),
            pl.BlockSpec((_GBLK, 3), lambda i: (i, 0)),
        ],
        out_specs=(
            pl.BlockSpec((1, _C1), lambda i: (0, 0)),
            pl.BlockSpec((1, _C1), lambda i: (0, 0)),
        ),
        out_shape=(
            jax.ShapeDtypeStruct((1, _C1), jnp.float32),
            jax.ShapeDtypeStruct((1, _C1), jnp.float32),
        ),
    )(g, w1xt, nxf)


def _mlp2_kernel(g_ref, w1xt_ref, nx_ref, s1_ref, ss1_ref, g1_ref, be1_ref,
                 w2t_ref, b2_ref, hmax_ref, hmin_ref, s2_ref, ss2_ref):
    i = pl.program_id(0)
    rinv = jnp.float32(1.0 / _R)
    m1 = s1_ref[...] * rinv
    v1 = ss1_ref[...] * rinv - m1 * m1
    sc1 = g1_ref[...] / jnp.sqrt(v1 + _EPS)
    nxw = lax.dot_general(nx_ref[...], w1xt_ref[...], (((1,), (0,)), ((), ())),
                          preferred_element_type=jnp.float32)
    h1 = (g_ref[...].reshape(_GBLK, _K, _C1)
          - nxw[:, None, :]).reshape(_RBLK, _C1)
    x = jnp.maximum((h1 - m1) * sc1 + be1_ref[...], 0.0)
    h2 = lax.dot_general(x, w2t_ref[...], (((1,), (0,)), ((), ())),
                         preferred_element_type=jnp.float32) + b2_ref[...]
    ps = jnp.sum(h2, axis=0, keepdims=True)
    pss = jnp.sum(h2 * h2, axis=0, keepdims=True)
    h3 = h2.reshape(_GBLK, _K, _C2)
    hmax_ref[...] = jnp.max(h3, axis=1)
    hmin_ref[...] = jnp.min(h3, axis=1)

    @pl.when(i == 0)
    def _():
        s2_ref[...] = ps
        ss2_ref[...] = pss

    @pl.when(i > 0)
    def _():
        s2_ref[...] += ps
        ss2_ref[...] += pss


def _run_mlp2(g, w1xt, nxf, s1, ss1, g1, be1, w2t, b2):
    return pl.pallas_call(
        _mlp2_kernel,
        grid=(_R // _RBLK,),
        in_specs=[
            pl.BlockSpec((_RBLK, _C1), lambda i: (i, 0)),
            pl.BlockSpec((3, _C1), lambda i: (0, 0)),
            pl.BlockSpec((_GBLK, 3), lambda i: (i, 0)),
            pl.BlockSpec((1, _C1), lambda i: (0, 0)),
            pl.BlockSpec((1, _C1), lambda i: (0, 0)),
            pl.BlockSpec((1, _C1), lambda i: (0, 0)),
            pl.BlockSpec((1, _C1), lambda i: (0, 0)),
            pl.BlockSpec((_C1, _C2), lambda i: (0, 0)),
            pl.BlockSpec((1, _C2), lambda i: (0, 0)),
        ],
        out_specs=(
            pl.BlockSpec((_GBLK, _C2), lambda i: (i, 0)),
            pl.BlockSpec((_GBLK, _C2), lambda i: (i, 0)),
            pl.BlockSpec((1, _C2), lambda i: (0, 0)),
            pl.BlockSpec((1, _C2), lambda i: (0, 0)),
        ),
        out_shape=(
            jax.ShapeDtypeStruct((_BS, _C2), jnp.float32),
            jax.ShapeDtypeStruct((_BS, _C2), jnp.float32),
            jax.ShapeDtypeStruct((1, _C2), jnp.float32),
            jax.ShapeDtypeStruct((1, _C2), jnp.float32),
        ),
    )(g, w1xt, nxf, s1, ss1, g1, be1, w2t, b2)


def _final_kernel(hmax_ref, hmin_ref, s2_ref, ss2_ref, g2_ref, be2_ref, o_ref):
    rinv = jnp.float32(1.0 / _R)
    m2 = s2_ref[...] * rinv
    v2 = ss2_ref[...] * rinv - m2 * m2
    g = g2_ref[...]
    sc2 = g / jnp.sqrt(v2 + _EPS)
    h = jnp.where(g >= 0.0, hmax_ref[...], hmin_ref[...])
    o_ref[...] = jnp.maximum((h - m2) * sc2 + be2_ref[...], 0.0)


def _run_final(hmax, hmin, s2, ss2, g2, be2):
    blk = 2048
    return pl.pallas_call(
        _final_kernel,
        grid=(_BS // blk,),
        in_specs=[
            pl.BlockSpec((blk, _C2), lambda i: (i, 0)),
            pl.BlockSpec((blk, _C2), lambda i: (i, 0)),
            pl.BlockSpec((1, _C2), lambda i: (0, 0)),
            pl.BlockSpec((1, _C2), lambda i: (0, 0)),
            pl.BlockSpec((1, _C2), lambda i: (0, 0)),
            pl.BlockSpec((1, _C2), lambda i: (0, 0)),
        ],
        out_specs=pl.BlockSpec((blk, _C2), lambda i: (i, 0)),
        out_shape=jax.ShapeDtypeStruct((_BS, _C2), jnp.float32),
    )(hmax, hmin, s2, ss2, g2, be2)


_gather_rows = _sc_gather


def kernel(xyz, points, W1, b1, gamma1, beta1, W2, b2, gamma2, beta2):
    xt = jnp.swapaxes(xyz, 1, 2)
    x, y, z = xt[:, 0], xt[:, 1], xt[:, 2]
    _, nx, ny, nz = _run_fps(x, y, z)
    new_xyz = jnp.stack([nx, ny, nz], axis=-1)

    gidx = _run_knn(new_xyz, xt)

    w1ft = W1[:, 3:].T
    w1xt = W1[:, :3].T
    xw = _run_ptrans(xyz.reshape(_TBL, 3), points.reshape(_TBL, _D),
                     w1xt, w1ft, b1.reshape(1, _C1))
    g = _gather_rows(xw, gidx.reshape(_R))

    nxf = new_xyz.reshape(_BS, 3)
    s1, ss1 = _run_stats1(g, w1xt, nxf)
    hmax, hmin, s2, ss2 = _run_mlp2(g, w1xt, nxf, s1, ss1,
                                    gamma1.reshape(1, _C1),
                                    beta1.reshape(1, _C1), W2.T,
                                    b2.reshape(1, _C2))

    out = _run_final(hmax, hmin, s2, ss2, gamma2.reshape(1, _C2),
                     beta2.reshape(1, _C2))
    return new_xyz, out.reshape(_B, _S, _C2)


# speedup vs baseline: 14.2263x; 1.0507x over previous
import functools

import jax
import jax.numpy as jnp
from jax import lax
from jax.experimental import pallas as pl
from jax.experimental.pallas import tpu as pltpu
from jax.experimental.pallas import tpu_sc as plsc

_B, _N, _S, _K, _D = 8, 4096, 1024, 16, 128
_R = _B * _S * _K
_BS = _B * _S
_TBL = _B * _N
_TC = 144
_C1 = 256
_C2 = 256
_EPS = 1e-5
_SBLK = 256
_RBLK = 2048
_GBLK = _RBLK // _K
_BIG = 1e30


def _fps_kernel(x_ref, y_ref, z_ref, idx_ref, nx_ref, ny_ref, nz_ref,
                dist_ref, lf_ref, bi_ref, bx_ref, by_ref, bz_ref):
    lane128 = lax.broadcasted_iota(jnp.int32, (_B, 128), 1)
    dist_ref[...] = jnp.full((_B, _N), 1e10, dtype=jnp.float32)
    lf_ref[...] = lax.broadcasted_iota(
        jnp.int32, (_B, _N), 1).astype(jnp.float32)

    def body(i, carry):
        f, cx, cy, cz = carry
        sel = lane128 == i
        seli = sel.astype(jnp.int32)
        self_ = sel.astype(jnp.float32)
        bi_ref[...] = bi_ref[...] + f * seli
        bx_ref[...] = bx_ref[...] + cx * self_
        by_ref[...] = by_ref[...] + cy * self_
        bz_ref[...] = bz_ref[...] + cz * self_
        dx = x_ref[...] - cx
        dy = y_ref[...] - cy
        dz = z_ref[...] - cz
        d = dx * dx + dy * dy + dz * dz
        du = jnp.minimum(dist_ref[...], d)
        dist_ref[...] = du

        td, tl = du, lf_ref[...]
        w = _N
        while w > 128:
            h = w // 2
            take = td[:, h:] > td[:, :h]
            td = jnp.where(take, td[:, h:], td[:, :h])
            tl = jnp.where(take, tl[:, h:], tl[:, :h])
            w = h
        m = jnp.max(td, axis=1, keepdims=True)
        fi = jnp.min(jnp.where(td == m, tl, float(_N)),
                     axis=1, keepdims=True)
        s2 = lf_ref[...] == fi
        cx2 = jnp.sum(jnp.where(s2, x_ref[...], 0.0), axis=1, keepdims=True)
        cy2 = jnp.sum(jnp.where(s2, y_ref[...], 0.0), axis=1, keepdims=True)
        cz2 = jnp.sum(jnp.where(s2, z_ref[...], 0.0), axis=1, keepdims=True)
        f2 = fi.astype(jnp.int32)
        return f2, cx2, cy2, cz2

    f = jnp.zeros((_B, 1), dtype=jnp.int32)
    cx = x_ref[:, 0:1]
    cy = y_ref[:, 0:1]
    cz = z_ref[:, 0:1]
    for o in range(_S // 128):
        bi_ref[...] = jnp.zeros((_B, 128), dtype=jnp.int32)
        bx_ref[...] = jnp.zeros((_B, 128), dtype=jnp.float32)
        by_ref[...] = jnp.zeros((_B, 128), dtype=jnp.float32)
        bz_ref[...] = jnp.zeros((_B, 128), dtype=jnp.float32)
        f, cx, cy, cz = lax.fori_loop(0, 128, body, (f, cx, cy, cz))
        idx_ref[:, o * 128:(o + 1) * 128] = bi_ref[...]
        nx_ref[:, o * 128:(o + 1) * 128] = bx_ref[...]
        ny_ref[:, o * 128:(o + 1) * 128] = by_ref[...]
        nz_ref[:, o * 128:(o + 1) * 128] = bz_ref[...]


def _run_fps(x, y, z):
    return pl.pallas_call(
        _fps_kernel,
        out_shape=(
            jax.ShapeDtypeStruct((_B, _S), jnp.int32),
            jax.ShapeDtypeStruct((_B, _S), jnp.float32),
            jax.ShapeDtypeStruct((_B, _S), jnp.float32),
            jax.ShapeDtypeStruct((_B, _S), jnp.float32),
        ),
        scratch_shapes=[
            pltpu.VMEM((_B, _N), jnp.float32),
            pltpu.VMEM((_B, _N), jnp.float32),
            pltpu.VMEM((_B, 128), jnp.int32),
            pltpu.VMEM((_B, 128), jnp.float32),
            pltpu.VMEM((_B, 128), jnp.float32),
            pltpu.VMEM((_B, 128), jnp.float32),
        ],
    )(x, y, z)


def _knn_kernel(nxyz_ref, xt_ref, idx_ref, dist_ref):
    b = pl.program_id(0)
    src = nxyz_ref[0]
    dst = xt_ref[0]
    rn = jnp.sum(src * src, axis=1, keepdims=True)
    cn = jnp.sum(dst * dst, axis=0, keepdims=True)
    m = lax.dot_general(src, dst, (((1,), (0,)), ((), ())),
                        preferred_element_type=jnp.float32)
    dist_ref[...] = rn + cn - 2.0 * m
    colf = lax.broadcasted_iota(
        jnp.int32, (_SBLK, _N), 1).astype(jnp.float32)
    base = b * _N
    for k in range(_K):
        d = dist_ref[...]
        mn = jnp.min(d, axis=1, keepdims=True)
        fj = jnp.min(jnp.where(d == mn, colf, float(_N)),
                     axis=1, keepdims=True)
        j = fj.astype(jnp.int32)
        idx_ref[0, :, k:k + 1] = j + base
        dist_ref[...] = jnp.where(colf == fj, _BIG, d)


def _run_knn(new_xyz, xt):
    return pl.pallas_call(
        _knn_kernel,
        grid=(_B, _S // _SBLK),
        in_specs=[
            pl.BlockSpec((1, _SBLK, 3), lambda b, j: (b, j, 0)),
            pl.BlockSpec((1, 3, _N), lambda b, j: (b, 0, 0)),
        ],
        out_specs=pl.BlockSpec((1, _SBLK, _K), lambda b, j: (b, j, 0)),
        out_shape=jax.ShapeDtypeStruct((_B, _S, _K), jnp.int32),
        scratch_shapes=[pltpu.VMEM((_SBLK, _N), jnp.float32)],
    )(new_xyz, xt)


def _sc_gather(table, gidx):
    info = plsc.get_sparse_core_info()
    nw = info.num_cores * info.num_subcores
    rows_per_w = _R // nw
    chunk = 128
    nchunk = rows_per_w // chunk
    mesh = plsc.VectorSubcoreMesh(core_axis_name="c", subcore_axis_name="s")

    @functools.partial(
        pl.kernel,
        mesh=mesh,
        out_type=jax.ShapeDtypeStruct((_R, _C1), jnp.float32),
        scratch_types=[
            pltpu.VMEM((chunk,), jnp.int32),
            pltpu.VMEM((chunk, _C1), jnp.float32),
            pltpu.SemaphoreType.DMA,
        ],
    )
    def k(table_hbm, gidx_hbm, out_hbm, idx_v, rows_v, sem):
        wid = lax.axis_index("s") * info.num_cores + lax.axis_index("c")
        base = wid * rows_per_w

        def body(c, carry):
            off = base + c * chunk
            pltpu.sync_copy(gidx_hbm.at[pl.ds(off, chunk)], idx_v)
            pltpu.async_copy(table_hbm.at[idx_v], rows_v, sem).wait()
            pltpu.sync_copy(rows_v, out_hbm.at[pl.ds(off, chunk)])
            return carry

        lax.fori_loop(0, nchunk, body, 0)

    return k(table, gidx)


def _ptrans_kernel(xyz_ref, pts_ref, w1xt_ref, w1ft_ref, b1_ref, o_ref):
    h = lax.dot_general(pts_ref[...], w1ft_ref[...], (((1,), (0,)), ((), ())),
                        preferred_element_type=jnp.float32)
    h = h + lax.dot_general(xyz_ref[...], w1xt_ref[...],
                            (((1,), (0,)), ((), ())),
                            preferred_element_type=jnp.float32)
    o_ref[...] = h + b1_ref[...]


def _run_ptrans(xyz_flat, pts_flat, w1xt, w1ft, b1):
    blk = 4096
    return pl.pallas_call(
        _ptrans_kernel,
        grid=(_TBL // blk,),
        in_specs=[
            pl.BlockSpec((blk, 3), lambda i: (i, 0)),
            pl.BlockSpec((blk, _D), lambda i: (i, 0)),
            pl.BlockSpec((3, _C1), lambda i: (0, 0)),
            pl.BlockSpec((_D, _C1), lambda i: (0, 0)),
            pl.BlockSpec((1, _C1), lambda i: (0, 0)),
        ],
        out_specs=pl.BlockSpec((blk, _C1), lambda i: (i, 0)),
        out_shape=jax.ShapeDtypeStruct((_TBL, _C1), jnp.float32),
    )(xyz_flat, pts_flat, w1xt, w1ft, b1)


def _stats1_kernel(g_ref, w1xt_ref, nx_ref, s_ref, ss_ref):
    i = pl.program_id(0)
    nxw = lax.dot_general(nx_ref[...], w1xt_ref[...], (((1,), (0,)), ((), ())),
                          preferred_element_type=jnp.float32)
    h3 = g_ref[...].reshape(_GBLK, _K, _C1) - nxw[:, None, :]
    ps = jnp.sum(h3, axis=(0, 1)).reshape(1, _C1)
    pss = jnp.sum(h3 * h3, axis=(0, 1)).reshape(1, _C1)

    @pl.when(i == 0)
    def _():
        s_ref[...] = ps
        ss_ref[...] = pss

    @pl.when(i > 0)
    def _():
        s_ref[...] += ps
        ss_ref[...] += pss


def _run_stats1(g, w1xt, nxf):
    return pl.pallas_call(
        _stats1_kernel,
        grid=(_R // _RBLK,),
        in_specs=[
            pl.BlockSpec((_RBLK, _C1), lambda i: (i, 0)),
            pl.BlockSpec((3, _C1), lambda i: (0, 0)),
            pl.BlockSpec((_GBLK, 3), lambda i: (i, 0)),
        ],
        out_specs=(
            pl.BlockSpec((1, _C1), lambda i: (0, 0)),
            pl.BlockSpec((1, _C1), lambda i: (0, 0)),
        ),
        out_shape=(
            jax.ShapeDtypeStruct((1, _C1), jnp.float32),
            jax.ShapeDtypeStruct((1, _C1), jnp.float32),
        ),
    )(g, w1xt, nxf)


def _mlp2_kernel(g_ref, w1xt_ref, nx_ref, s1_ref, ss1_ref, g1_ref, be1_ref,
                 w2t_ref, b2_ref, hmax_ref, hmin_ref, s2_ref, ss2_ref):
    i = pl.program_id(0)
    rinv = jnp.float32(1.0 / _R)
    m1 = s1_ref[...] * rinv
    v1 = ss1_ref[...] * rinv - m1 * m1
    sc1 = g1_ref[...] / jnp.sqrt(v1 + _EPS)
    nxw = lax.dot_general(nx_ref[...], w1xt_ref[...], (((1,), (0,)), ((), ())),
                          preferred_element_type=jnp.float32)
    h1 = (g_ref[...].reshape(_GBLK, _K, _C1)
          - nxw[:, None, :]).reshape(_RBLK, _C1)
    x = jnp.maximum((h1 - m1) * sc1 + be1_ref[...], 0.0)
    h2 = lax.dot_general(x, w2t_ref[...], (((1,), (0,)), ((), ())),
                         preferred_element_type=jnp.float32) + b2_ref[...]
    ps = jnp.sum(h2, axis=0, keepdims=True)
    pss = jnp.sum(h2 * h2, axis=0, keepdims=True)
    h3 = h2.reshape(_GBLK, _K, _C2)
    hmax_ref[...] = jnp.max(h3, axis=1)
    hmin_ref[...] = jnp.min(h3, axis=1)

    @pl.when(i == 0)
    def _():
        s2_ref[...] = ps
        ss2_ref[...] = pss

    @pl.when(i > 0)
    def _():
        s2_ref[...] += ps
        ss2_ref[...] += pss


def _run_mlp2(g, w1xt, nxf, s1, ss1, g1, be1, w2t, b2):
    return pl.pallas_call(
        _mlp2_kernel,
        grid=(_R // _RBLK,),
        in_specs=[
            pl.BlockSpec((_RBLK, _C1), lambda i: (i, 0)),
            pl.BlockSpec((3, _C1), lambda i: (0, 0)),
            pl.BlockSpec((_GBLK, 3), lambda i: (i, 0)),
            pl.BlockSpec((1, _C1), lambda i: (0, 0)),
            pl.BlockSpec((1, _C1), lambda i: (0, 0)),
            pl.BlockSpec((1, _C1), lambda i: (0, 0)),
            pl.BlockSpec((1, _C1), lambda i: (0, 0)),
            pl.BlockSpec((_C1, _C2), lambda i: (0, 0)),
            pl.BlockSpec((1, _C2), lambda i: (0, 0)),
        ],
        out_specs=(
            pl.BlockSpec((_GBLK, _C2), lambda i: (i, 0)),
            pl.BlockSpec((_GBLK, _C2), lambda i: (i, 0)),
            pl.BlockSpec((1, _C2), lambda i: (0, 0)),
            pl.BlockSpec((1, _C2), lambda i: (0, 0)),
        ),
        out_shape=(
            jax.ShapeDtypeStruct((_BS, _C2), jnp.float32),
            jax.ShapeDtypeStruct((_BS, _C2), jnp.float32),
            jax.ShapeDtypeStruct((1, _C2), jnp.float32),
            jax.ShapeDtypeStruct((1, _C2), jnp.float32),
        ),
    )(g, w1xt, nxf, s1, ss1, g1, be1, w2t, b2)


def _final_kernel(hmax_ref, hmin_ref, s2_ref, ss2_ref, g2_ref, be2_ref, o_ref):
    rinv = jnp.float32(1.0 / _R)
    m2 = s2_ref[...] * rinv
    v2 = ss2_ref[...] * rinv - m2 * m2
    g = g2_ref[...]
    sc2 = g / jnp.sqrt(v2 + _EPS)
    h = jnp.where(g >= 0.0, hmax_ref[...], hmin_ref[...])
    o_ref[...] = jnp.maximum((h - m2) * sc2 + be2_ref[...], 0.0)


def _run_final(hmax, hmin, s2, ss2, g2, be2):
    blk = 2048
    return pl.pallas_call(
        _final_kernel,
        grid=(_BS // blk,),
        in_specs=[
            pl.BlockSpec((blk, _C2), lambda i: (i, 0)),
            pl.BlockSpec((blk, _C2), lambda i: (i, 0)),
            pl.BlockSpec((1, _C2), lambda i: (0, 0)),
            pl.BlockSpec((1, _C2), lambda i: (0, 0)),
            pl.BlockSpec((1, _C2), lambda i: (0, 0)),
            pl.BlockSpec((1, _C2), lambda i: (0, 0)),
        ],
        out_specs=pl.BlockSpec((blk, _C2), lambda i: (i, 0)),
        out_shape=jax.ShapeDtypeStruct((_BS, _C2), jnp.float32),
    )(hmax, hmin, s2, ss2, g2, be2)


_gather_rows = _sc_gather


def kernel(xyz, points, W1, b1, gamma1, beta1, W2, b2, gamma2, beta2):
    xt = jnp.swapaxes(xyz, 1, 2)
    x, y, z = xt[:, 0], xt[:, 1], xt[:, 2]
    _, nx, ny, nz = _run_fps(x, y, z)
    new_xyz = jnp.stack([nx, ny, nz], axis=-1)

    gidx = _run_knn(new_xyz, xt)

    w1ft = W1[:, 3:].T
    w1xt = W1[:, :3].T
    xw = _run_ptrans(xyz.reshape(_TBL, 3), points.reshape(_TBL, _D),
                     w1xt, w1ft, b1.reshape(1, _C1))
    g = _gather_rows(xw, gidx.reshape(_R))

    nxf = new_xyz.reshape(_BS, 3)
    s1, ss1 = _run_stats1(g, w1xt, nxf)
    hmax, hmin, s2, ss2 = _run_mlp2(g, w1xt, nxf, s1, ss1,
                                    gamma1.reshape(1, _C1),
                                    beta1.reshape(1, _C1), W2.T,
                                    b2.reshape(1, _C2))

    out = _run_final(hmax, hmin, s2, ss2, gamma2.reshape(1, _C2),
                     beta2.reshape(1, _C2))
    return new_xyz, out.reshape(_B, _S, _C2)


# speedup vs baseline: 14.5604x; 1.0235x over previous
import functools

import jax
import jax.numpy as jnp
from jax import lax
from jax.experimental import pallas as pl
from jax.experimental.pallas import tpu as pltpu
from jax.experimental.pallas import tpu_sc as plsc

_B, _N, _S, _K, _D = 8, 4096, 1024, 16, 128
_R = _B * _S * _K
_BS = _B * _S
_TBL = _B * _N
_TC = 144
_C1 = 256
_C2 = 256
_EPS = 1e-5
_SBLK = 256
_RBLK = 2048
_GBLK = _RBLK // _K
_BIG = 1e30


def _fps_kernel(x_ref, y_ref, z_ref, idx_ref, nx_ref, ny_ref, nz_ref,
                dist_ref, lf_ref, bi_ref, bx_ref, by_ref, bz_ref):
    lane128 = lax.broadcasted_iota(jnp.int32, (_B, 128), 1)
    dist_ref[...] = jnp.full((_B, _N), 1e10, dtype=jnp.float32)
    lf_ref[...] = lax.broadcasted_iota(
        jnp.int32, (_B, _N), 1).astype(jnp.float32)

    def body(i, carry):
        f, cx, cy, cz = carry
        sel = lane128 == i
        seli = sel.astype(jnp.int32)
        self_ = sel.astype(jnp.float32)
        bi_ref[...] = bi_ref[...] + f * seli
        bx_ref[...] = bx_ref[...] + cx * self_
        by_ref[...] = by_ref[...] + cy * self_
        bz_ref[...] = bz_ref[...] + cz * self_
        dx = x_ref[...] - cx
        dy = y_ref[...] - cy
        dz = z_ref[...] - cz
        d = dx * dx + dy * dy + dz * dz
        du = jnp.minimum(dist_ref[...], d)
        dist_ref[...] = du

        td, tl = du, lf_ref[...]
        tx, ty, tz = x_ref[...], y_ref[...], z_ref[...]
        w = _N
        while w > 128:
            h = w // 2
            take = td[:, h:] > td[:, :h]
            td = jnp.where(take, td[:, h:], td[:, :h])
            tl = jnp.where(take, tl[:, h:], tl[:, :h])
            tx = jnp.where(take, tx[:, h:], tx[:, :h])
            ty = jnp.where(take, ty[:, h:], ty[:, :h])
            tz = jnp.where(take, tz[:, h:], tz[:, :h])
            w = h
        m = jnp.max(td, axis=1, keepdims=True)
        fi = jnp.min(jnp.where(td == m, tl, float(_N)),
                     axis=1, keepdims=True)
        s2 = tl == fi
        cx2 = jnp.sum(jnp.where(s2, tx, 0.0), axis=1, keepdims=True)
        cy2 = jnp.sum(jnp.where(s2, ty, 0.0), axis=1, keepdims=True)
        cz2 = jnp.sum(jnp.where(s2, tz, 0.0), axis=1, keepdims=True)
        f2 = fi.astype(jnp.int32)
        return f2, cx2, cy2, cz2

    f = jnp.zeros((_B, 1), dtype=jnp.int32)
    cx = x_ref[:, 0:1]
    cy = y_ref[:, 0:1]
    cz = z_ref[:, 0:1]
    for o in range(_S // 128):
        bi_ref[...] = jnp.zeros((_B, 128), dtype=jnp.int32)
        bx_ref[...] = jnp.zeros((_B, 128), dtype=jnp.float32)
        by_ref[...] = jnp.zeros((_B, 128), dtype=jnp.float32)
        bz_ref[...] = jnp.zeros((_B, 128), dtype=jnp.float32)
        f, cx, cy, cz = lax.fori_loop(0, 128, body, (f, cx, cy, cz))
        idx_ref[:, o * 128:(o + 1) * 128] = bi_ref[...]
        nx_ref[:, o * 128:(o + 1) * 128] = bx_ref[...]
        ny_ref[:, o * 128:(o + 1) * 128] = by_ref[...]
        nz_ref[:, o * 128:(o + 1) * 128] = bz_ref[...]


def _run_fps(x, y, z):
    return pl.pallas_call(
        _fps_kernel,
        out_shape=(
            jax.ShapeDtypeStruct((_B, _S), jnp.int32),
            jax.ShapeDtypeStruct((_B, _S), jnp.float32),
            jax.ShapeDtypeStruct((_B, _S), jnp.float32),
            jax.ShapeDtypeStruct((_B, _S), jnp.float32),
        ),
        scratch_shapes=[
            pltpu.VMEM((_B, _N), jnp.float32),
            pltpu.VMEM((_B, _N), jnp.float32),
            pltpu.VMEM((_B, 128), jnp.int32),
            pltpu.VMEM((_B, 128), jnp.float32),
            pltpu.VMEM((_B, 128), jnp.float32),
            pltpu.VMEM((_B, 128), jnp.float32),
        ],
    )(x, y, z)


def _knn_kernel(nxyz_ref, xt_ref, idx_ref, dist_ref):
    b = pl.program_id(0)
    src = nxyz_ref[0]
    dst = xt_ref[0]
    rn = jnp.sum(src * src, axis=1, keepdims=True)
    cn = jnp.sum(dst * dst, axis=0, keepdims=True)
    m = lax.dot_general(src, dst, (((1,), (0,)), ((), ())),
                        preferred_element_type=jnp.float32)
    dist_ref[...] = rn + cn - 2.0 * m
    colf = lax.broadcasted_iota(
        jnp.int32, (_SBLK, _N), 1).astype(jnp.float32)
    base = b * _N
    for k in range(_K):
        d = dist_ref[...]
        mn = jnp.min(d, axis=1, keepdims=True)
        fj = jnp.min(jnp.where(d == mn, colf, float(_N)),
                     axis=1, keepdims=True)
        j = fj.astype(jnp.int32)
        idx_ref[0, :, k:k + 1] = j + base
        dist_ref[...] = jnp.where(colf == fj, _BIG, d)


def _run_knn(new_xyz, xt):
    return pl.pallas_call(
        _knn_kernel,
        grid=(_B, _S // _SBLK),
        in_specs=[
            pl.BlockSpec((1, _SBLK, 3), lambda b, j: (b, j, 0)),
            pl.BlockSpec((1, 3, _N), lambda b, j: (b, 0, 0)),
        ],
        out_specs=pl.BlockSpec((1, _SBLK, _K), lambda b, j: (b, j, 0)),
        out_shape=jax.ShapeDtypeStruct((_B, _S, _K), jnp.int32),
        scratch_shapes=[pltpu.VMEM((_SBLK, _N), jnp.float32)],
    )(new_xyz, xt)


def _sc_gather(table, gidx):
    info = plsc.get_sparse_core_info()
    nw = info.num_cores * info.num_subcores
    rows_per_w = _R // nw
    chunk = 128
    nchunk = rows_per_w // chunk
    mesh = plsc.VectorSubcoreMesh(core_axis_name="c", subcore_axis_name="s")

    @functools.partial(
        pl.kernel,
        mesh=mesh,
        out_type=jax.ShapeDtypeStruct((_R, _C1), jnp.float32),
        scratch_types=[
            pltpu.VMEM((chunk,), jnp.int32),
            pltpu.VMEM((chunk, _C1), jnp.float32),
            pltpu.SemaphoreType.DMA,
        ],
    )
    def k(table_hbm, gidx_hbm, out_hbm, idx_v, rows_v, sem):
        wid = lax.axis_index("s") * info.num_cores + lax.axis_index("c")
        base = wid * rows_per_w

        def body(c, carry):
            off = base + c * chunk
            pltpu.sync_copy(gidx_hbm.at[pl.ds(off, chunk)], idx_v)
            pltpu.async_copy(table_hbm.at[idx_v], rows_v, sem).wait()
            pltpu.sync_copy(rows_v, out_hbm.at[pl.ds(off, chunk)])
            return carry

        lax.fori_loop(0, nchunk, body, 0)

    return k(table, gidx)


def _ptrans_kernel(xyz_ref, pts_ref, w1xt_ref, w1ft_ref, b1_ref, o_ref):
    h = lax.dot_general(pts_ref[...], w1ft_ref[...], (((1,), (0,)), ((), ())),
                        preferred_element_type=jnp.float32)
    h = h + lax.dot_general(xyz_ref[...], w1xt_ref[...],
                            (((1,), (0,)), ((), ())),
                            preferred_element_type=jnp.float32)
    o_ref[...] = h + b1_ref[...]


def _run_ptrans(xyz_flat, pts_flat, w1xt, w1ft, b1):
    blk = 4096
    return pl.pallas_call(
        _ptrans_kernel,
        grid=(_TBL // blk,),
        in_specs=[
            pl.BlockSpec((blk, 3), lambda i: (i, 0)),
            pl.BlockSpec((blk, _D), lambda i: (i, 0)),
            pl.BlockSpec((3, _C1), lambda i: (0, 0)),
            pl.BlockSpec((_D, _C1), lambda i: (0, 0)),
            pl.BlockSpec((1, _C1), lambda i: (0, 0)),
        ],
        out_specs=pl.BlockSpec((blk, _C1), lambda i: (i, 0)),
        out_shape=jax.ShapeDtypeStruct((_TBL, _C1), jnp.float32),
    )(xyz_flat, pts_flat, w1xt, w1ft, b1)


def _stats1_kernel(g_ref, w1xt_ref, nx_ref, s_ref, ss_ref):
    i = pl.program_id(0)
    nxw = lax.dot_general(nx_ref[...], w1xt_ref[...], (((1,), (0,)), ((), ())),
                          preferred_element_type=jnp.float32)
    h3 = g_ref[...].reshape(_GBLK, _K, _C1) - nxw[:, None, :]
    ps = jnp.sum(h3, axis=(0, 1)).reshape(1, _C1)
    pss = jnp.sum(h3 * h3, axis=(0, 1)).reshape(1, _C1)

    @pl.when(i == 0)
    def _():
        s_ref[...] = ps
        ss_ref[...] = pss

    @pl.when(i > 0)
    def _():
        s_ref[...] += ps
        ss_ref[...] += pss


def _run_stats1(g, w1xt, nxf):
    return pl.pallas_call(
        _stats1_kernel,
        grid=(_R // _RBLK,),
        in_specs=[
            pl.BlockSpec((_RBLK, _C1), lambda i: (i, 0)),
            pl.BlockSpec((3, _C1), lambda i: (0, 0)),
            pl.BlockSpec((_GBLK, 3), lambda i: (i, 0)),
        ],
        out_specs=(
            pl.BlockSpec((1, _C1), lambda i: (0, 0)),
            pl.BlockSpec((1, _C1), lambda i: (0, 0)),
        ),
        out_shape=(
            jax.ShapeDtypeStruct((1, _C1), jnp.float32),
            jax.ShapeDtypeStruct((1, _C1), jnp.float32),
        ),
    )(g, w1xt, nxf)


def _mlp2_kernel(g_ref, w1xt_ref, nx_ref, s1_ref, ss1_ref, g1_ref, be1_ref,
                 w2t_ref, b2_ref, hmax_ref, hmin_ref, s2_ref, ss2_ref):
    i = pl.program_id(0)
    rinv = jnp.float32(1.0 / _R)
    m1 = s1_ref[...] * rinv
    v1 = ss1_ref[...] * rinv - m1 * m1
    sc1 = g1_ref[...] / jnp.sqrt(v1 + _EPS)
    nxw = lax.dot_general(nx_ref[...], w1xt_ref[...], (((1,), (0,)), ((), ())),
                          preferred_element_type=jnp.float32)
    h1 = (g_ref[...].reshape(_GBLK, _K, _C1)
          - nxw[:, None, :]).reshape(_RBLK, _C1)
    x = jnp.maximum((h1 - m1) * sc1 + be1_ref[...], 0.0)
    h2 = lax.dot_general(x, w2t_ref[...], (((1,), (0,)), ((), ())),
                         preferred_element_type=jnp.float32) + b2_ref[...]
    ps = jnp.sum(h2, axis=0, keepdims=True)
    pss = jnp.sum(h2 * h2, axis=0, keepdims=True)
    h3 = h2.reshape(_GBLK, _K, _C2)
    hmax_ref[...] = jnp.max(h3, axis=1)
    hmin_ref[...] = jnp.min(h3, axis=1)

    @pl.when(i == 0)
    def _():
        s2_ref[...] = ps
        ss2_ref[...] = pss

    @pl.when(i > 0)
    def _():
        s2_ref[...] += ps
        ss2_ref[...] += pss


def _run_mlp2(g, w1xt, nxf, s1, ss1, g1, be1, w2t, b2):
    return pl.pallas_call(
        _mlp2_kernel,
        grid=(_R // _RBLK,),
        in_specs=[
            pl.BlockSpec((_RBLK, _C1), lambda i: (i, 0)),
            pl.BlockSpec((3, _C1), lambda i: (0, 0)),
            pl.BlockSpec((_GBLK, 3), lambda i: (i, 0)),
            pl.BlockSpec((1, _C1), lambda i: (0, 0)),
            pl.BlockSpec((1, _C1), lambda i: (0, 0)),
            pl.BlockSpec((1, _C1), lambda i: (0, 0)),
            pl.BlockSpec((1, _C1), lambda i: (0, 0)),
            pl.BlockSpec((_C1, _C2), lambda i: (0, 0)),
            pl.BlockSpec((1, _C2), lambda i: (0, 0)),
        ],
        out_specs=(
            pl.BlockSpec((_GBLK, _C2), lambda i: (i, 0)),
            pl.BlockSpec((_GBLK, _C2), lambda i: (i, 0)),
            pl.BlockSpec((1, _C2), lambda i: (0, 0)),
            pl.BlockSpec((1, _C2), lambda i: (0, 0)),
        ),
        out_shape=(
            jax.ShapeDtypeStruct((_BS, _C2), jnp.float32),
            jax.ShapeDtypeStruct((_BS, _C2), jnp.float32),
            jax.ShapeDtypeStruct((1, _C2), jnp.float32),
            jax.ShapeDtypeStruct((1, _C2), jnp.float32),
        ),
    )(g, w1xt, nxf, s1, ss1, g1, be1, w2t, b2)


def _final_kernel(hmax_ref, hmin_ref, s2_ref, ss2_ref, g2_ref, be2_ref, o_ref):
    rinv = jnp.float32(1.0 / _R)
    m2 = s2_ref[...] * rinv
    v2 = ss2_ref[...] * rinv - m2 * m2
    g = g2_ref[...]
    sc2 = g / jnp.sqrt(v2 + _EPS)
    h = jnp.where(g >= 0.0, hmax_ref[...], hmin_ref[...])
    o_ref[...] = jnp.maximum((h - m2) * sc2 + be2_ref[...], 0.0)


def _run_final(hmax, hmin, s2, ss2, g2, be2):
    blk = 2048
    return pl.pallas_call(
        _final_kernel,
        grid=(_BS // blk,),
        in_specs=[
            pl.BlockSpec((blk, _C2), lambda i: (i, 0)),
            pl.BlockSpec((blk, _C2), lambda i: (i, 0)),
            pl.BlockSpec((1, _C2), lambda i: (0, 0)),
            pl.BlockSpec((1, _C2), lambda i: (0, 0)),
            pl.BlockSpec((1, _C2), lambda i: (0, 0)),
            pl.BlockSpec((1, _C2), lambda i: (0, 0)),
        ],
        out_specs=pl.BlockSpec((blk, _C2), lambda i: (i, 0)),
        out_shape=jax.ShapeDtypeStruct((_BS, _C2), jnp.float32),
    )(hmax, hmin, s2, ss2, g2, be2)


_gather_rows = _sc_gather


def kernel(xyz, points, W1, b1, gamma1, beta1, W2, b2, gamma2, beta2):
    xt = jnp.swapaxes(xyz, 1, 2)
    x, y, z = xt[:, 0], xt[:, 1], xt[:, 2]
    _, nx, ny, nz = _run_fps(x, y, z)
    new_xyz = jnp.stack([nx, ny, nz], axis=-1)

    gidx = _run_knn(new_xyz, xt)

    w1ft = W1[:, 3:].T
    w1xt = W1[:, :3].T
    xw = _run_ptrans(xyz.reshape(_TBL, 3), points.reshape(_TBL, _D),
                     w1xt, w1ft, b1.reshape(1, _C1))
    g = _gather_rows(xw, gidx.reshape(_R))

    nxf = new_xyz.reshape(_BS, 3)
    s1, ss1 = _run_stats1(g, w1xt, nxf)
    hmax, hmin, s2, ss2 = _run_mlp2(g, w1xt, nxf, s1, ss1,
                                    gamma1.reshape(1, _C1),
                                    beta1.reshape(1, _C1), W2.T,
                                    b2.reshape(1, _C2))

    out = _run_final(hmax, hmin, s2, ss2, gamma2.reshape(1, _C2),
                     beta2.reshape(1, _C2))
    return new_xyz, out.reshape(_B, _S, _C2)
